# baseline (speedup 1.0000x reference)
import functools

import jax
import jax.numpy as jnp
from jax import lax
from jax.experimental import pallas as pl
from jax.experimental.pallas import tpu as pltpu
from jax.experimental.pallas import tpu_sc as plsc

B = 4096
F = 26
V = 100000
D = 32
NUM = 13

_info = plsc.get_sparse_core_info()
_NC, _NS = _info.num_cores, _info.num_subcores
_NW = _NC * _NS

_PACK = 128 // D
_PR = F * V // _PACK
_R = B * F
_RPW = _R // _NW
_CHUNK = 128
_NCHUNK = _RPW // _CHUNK
_NPAIR = _NCHUNK // 2


def _gather_body(table_hbm, pidx_hbm, col_hbm, out_hbm,
                 pidx_v, col_v, buf0, buf1, stg0, stg1,
                 sem0, sem1, wsem0, wsem1):
    wid = lax.axis_index("s") * _NC + lax.axis_index("c")
    obase = wid * _RPW * D
    pltpu.sync_copy(pidx_hbm.at[wid], pidx_v)
    pltpu.sync_copy(col_hbm.at[wid], col_v)

    def fire(j, buf, sem):
        return pltpu.make_async_copy(table_hbm.at[pidx_v.at[j]], buf, sem)

    def fire_wb(j, stg, wsem):
        return pltpu.make_async_copy(
            stg, out_hbm.at[pl.ds(obase + j * _CHUNK * D, _CHUNK * D)], wsem)

    def extract(j, buf, stg):
        def group(g, _):
            colv = col_v[j, pl.ds(g * 16, 16)]
            for l in range(16):
                i = g * 16 + l
                o = colv[l]
                for h in range(D // 16):
                    stg[pl.ds(i * D + h * 16, 16)] = buf[i, pl.ds(o + h * 16, 16)]
            return 0
        lax.fori_loop(0, _CHUNK // 16, group, 0, unroll=False)

    fire(0, buf0, sem0).start()

    def pair(k, _):
        j0 = 2 * k
        fire(j0 + 1, buf1, sem1).start()
        fire(j0, buf0, sem0).wait()

        @pl.when(k > 0)
        def _():
            fire_wb(j0 - 2, stg0, wsem0).wait()
        extract(j0, buf0, stg0)
        fire_wb(j0, stg0, wsem0).start()

        @pl.when(k < _NPAIR - 1)
        def _():
            fire(j0 + 2, buf0, sem0).start()
        fire(j0 + 1, buf1, sem1).wait()

        @pl.when(k > 0)
        def _():
            fire_wb(j0 - 1, stg1, wsem1).wait()
        extract(j0 + 1, buf1, stg1)
        fire_wb(j0 + 1, stg1, wsem1).start()
        return 0

    lax.fori_loop(0, _NPAIR, pair, 0, unroll=False)
    fire_wb(_NCHUNK - 2, stg0, wsem0).wait()
    fire_wb(_NCHUNK - 1, stg1, wsem1).wait()


_gather_call = functools.partial(
    pl.kernel,
    mesh=plsc.VectorSubcoreMesh(core_axis_name="c", subcore_axis_name="s"),
    out_type=jax.ShapeDtypeStruct((_R * D,), jnp.float32),
    scratch_types=[
        pltpu.VMEM((_NCHUNK, _CHUNK), jnp.int32),
        pltpu.VMEM((_NCHUNK, _CHUNK), jnp.int32),
        pltpu.VMEM((_CHUNK, 128), jnp.float32),
        pltpu.VMEM((_CHUNK, 128), jnp.float32),
        pltpu.VMEM((_CHUNK * D,), jnp.float32),
        pltpu.VMEM((_CHUNK * D,), jnp.float32),
        pltpu.SemaphoreType.DMA,
        pltpu.SemaphoreType.DMA,
        pltpu.SemaphoreType.DMA,
        pltpu.SemaphoreType.DMA,
    ],
)(_gather_body)


def _mlp_body(emb_ref, num_ref, w0e_ref, w0n_ref, b0_ref, w1_ref, b1_ref,
              w2_ref, b2_ref, w3_ref, b3_ref, out_ref):
    h = jnp.dot(emb_ref[...], w0e_ref[...], preferred_element_type=jnp.float32)
    h = h + jnp.dot(num_ref[...], w0n_ref[...], preferred_element_type=jnp.float32)
    h = jnp.maximum(h + b0_ref[...], 0.0)
    h = jnp.maximum(jnp.dot(h, w1_ref[...], preferred_element_type=jnp.float32) + b1_ref[...], 0.0)
    h = jnp.maximum(jnp.dot(h, w2_ref[...], preferred_element_type=jnp.float32) + b2_ref[...], 0.0)
    out_ref[...] = jnp.dot(h, w3_ref[...], preferred_element_type=jnp.float32) + b3_ref[...]


_BB = 512


def _mlp_call(emb, num, w0e, w0n, b0, w1, b1, w2, b2, w3, b3):
    full = lambda shape: pl.BlockSpec(shape, lambda i: (0, 0))
    return pl.pallas_call(
        _mlp_body,
        grid=(B // _BB,),
        in_specs=[
            pl.BlockSpec((_BB, F * D), lambda i: (i, 0)),
            pl.BlockSpec((_BB, NUM), lambda i: (i, 0)),
            full(w0e.shape), full(w0n.shape), full(b0.shape),
            full(w1.shape), full(b1.shape),
            full(w2.shape), full(b2.shape),
            full(w3.shape), full(b3.shape),
        ],
        out_specs=pl.BlockSpec((_BB, 1), lambda i: (i, 0)),
        out_shape=jax.ShapeDtypeStruct((B, 1), jnp.float32),
    )(emb, num, w0e, w0n, b0, w1, b1, w2, b2, w3, b3)


@jax.jit
def kernel(numerical_features, cat_features, tables, W0, b0, W1, b1, W2, b2, W3, b3):
    table_packed = tables.reshape(_PR, 128)
    gidx = (cat_features + jnp.arange(F, dtype=jnp.int32)[None, :] * V).reshape(
        _NW, _NCHUNK, _CHUNK)
    pidx = gidx >> 2
    col = (gidx & 3) * D
    flat = _gather_call(table_packed, pidx, col)
    emb = flat.reshape(B, F * D)
    out = _mlp_call(
        emb, numerical_features,
        W0[:F * D], W0[F * D:], b0.reshape(1, -1),
        W1, b1.reshape(1, -1), W2, b2.reshape(1, -1), W3, b3.reshape(1, -1),
    )
    return out


# speedup vs baseline: 5.1122x; 5.1122x over previous
import functools

import jax
import jax.numpy as jnp
from jax import lax
from jax.experimental import pallas as pl
from jax.experimental.pallas import tpu as pltpu
from jax.experimental.pallas import tpu_sc as plsc

B = 4096
F = 26
V = 100000
D = 32
NUM = 13

_info = plsc.get_sparse_core_info()
_NC, _NS = _info.num_cores, _info.num_subcores
_NW = _NC * _NS


def _gather_body(tt_hbm, cidx_hbm, out_hbm, lane_v, idx_v, ob_v):
    w = lax.axis_index("s") * _NC + lax.axis_index("c")

    def per_field(f, _):
        pltpu.sync_copy(cidx_hbm.at[f], idx_v)
        pltpu.sync_copy(tt_hbm.at[f].at[w], lane_v)

        def group(g, _):
            vb = idx_v[pl.ds(g * 16, 16)]
            ob_v[pl.ds(g * 16, 16)] = plsc.load_gather(lane_v, [vb])
            return 0

        lax.fori_loop(0, B // 16, group, 0, unroll=8)
        pltpu.sync_copy(ob_v, out_hbm.at[f * D + w])
        return 0

    lax.fori_loop(0, F, per_field, 0, unroll=False)


_gather_call = functools.partial(
    pl.kernel,
    mesh=plsc.VectorSubcoreMesh(core_axis_name="c", subcore_axis_name="s"),
    out_type=jax.ShapeDtypeStruct((F * D, B), jnp.float32),
    compiler_params=pltpu.CompilerParams(needs_layout_passes=False),
    scratch_types=[
        pltpu.VMEM((V,), jnp.float32),
        pltpu.VMEM((B,), jnp.int32),
        pltpu.VMEM((B,), jnp.float32),
    ],
)(_gather_body)


def _mlp_body(embT_ref, numT_ref, w0eT_ref, w0nT_ref, b0_ref, w1T_ref, b1_ref,
              w2T_ref, b2_ref, w3T_ref, b3_ref, out_ref):
    h = jnp.dot(w0eT_ref[...], embT_ref[...], preferred_element_type=jnp.float32)
    h = h + jnp.dot(w0nT_ref[...], numT_ref[...], preferred_element_type=jnp.float32)
    h = jnp.maximum(h + b0_ref[...], 0.0)
    h = jnp.maximum(jnp.dot(w1T_ref[...], h, preferred_element_type=jnp.float32) + b1_ref[...], 0.0)
    h = jnp.maximum(jnp.dot(w2T_ref[...], h, preferred_element_type=jnp.float32) + b2_ref[...], 0.0)
    out_ref[...] = jnp.dot(w3T_ref[...], h, preferred_element_type=jnp.float32) + b3_ref[...]


_BB = 512


def _mlp_call(embT, numT, w0eT, w0nT, b0, w1T, b1, w2T, b2, w3T, b3):
    full = lambda shape: pl.BlockSpec(shape, lambda i: (0, 0))
    return pl.pallas_call(
        _mlp_body,
        grid=(B // _BB,),
        in_specs=[
            pl.BlockSpec((F * D, _BB), lambda i: (0, i)),
            pl.BlockSpec((NUM, _BB), lambda i: (0, i)),
            full(w0eT.shape), full(w0nT.shape), full(b0.shape),
            full(w1T.shape), full(b1.shape),
            full(w2T.shape), full(b2.shape),
            full(w3T.shape), full(b3.shape),
        ],
        out_specs=pl.BlockSpec((1, _BB), lambda i: (0, i)),
        out_shape=jax.ShapeDtypeStruct((1, B), jnp.float32),
    )(embT, numT, w0eT, w0nT, b0, w1T, b1, w2T, b2, w3T, b3)


@jax.jit
def kernel(numerical_features, cat_features, tables, W0, b0, W1, b1, W2, b2, W3, b3):
    tt = tables.transpose(0, 2, 1)
    cidx = cat_features.T
    embT = _gather_call(tt, cidx)
    outT = _mlp_call(
        embT, numerical_features.T,
        W0[:F * D].T, W0[F * D:].T, b0.reshape(-1, 1),
        W1.T, b1.reshape(-1, 1), W2.T, b2.reshape(-1, 1), W3.T, b3.reshape(-1, 1),
    )
    return outT.reshape(B, 1)


# speedup vs baseline: 5.7523x; 1.1252x over previous
import functools

import jax
import jax.numpy as jnp
from jax import lax
from jax.experimental import pallas as pl
from jax.experimental.pallas import tpu as pltpu
from jax.experimental.pallas import tpu_sc as plsc

B = 4096
F = 26
V = 100000
D = 32
NUM = 13

_info = plsc.get_sparse_core_info()
_NC, _NS = _info.num_cores, _info.num_subcores
_NW = _NC * _NS


_H0 = 50048
_H1 = V - _H0
_H1A = 49920
_H1T = _H1 - _H1A
_NPAIR = F // 2


def _gather_body(tt_hbm, tail_hbm, cidx_hbm, out_hbm,
                 lane0, lane1, idxA, idxB, obA, obB,
                 sl0, sl1, siA, siB, soA, soB):
    w = lax.axis_index("s") * _NC + lax.axis_index("c")

    def fire_idx(f, idx_v, sem):
        return pltpu.make_async_copy(cidx_hbm.at[f], idx_v, sem)

    def fire_l0(f, sem):
        return pltpu.make_async_copy(
            tt_hbm.at[f].at[w].at[pl.ds(0, _H0)], lane0, sem)

    def fire_l1(f, sem):
        return pltpu.make_async_copy(
            tt_hbm.at[f].at[w].at[pl.ds(_H0, _H1A)],
            lane1.at[pl.ds(0, _H1A)], sem)

    def fire_l1t(f, sem):
        return pltpu.make_async_copy(
            tail_hbm.at[f].at[w], lane1.at[pl.ds(_H1A, 128)], sem)

    def fire_out(f, ob, sem):
        return pltpu.make_async_copy(ob, out_hbm.at[f * D + w], sem)

    def pass0(idx_v, ob):
        def grp(g, _):
            vb = idx_v[pl.ds(g * 16, 16)]
            m = vb < _H0
            vals = plsc.load_gather(lane0, [jnp.minimum(vb, _H0 - 1)])
            ob[pl.ds(g * 16, 16)] = jnp.where(m, vals, 0.0)
            return 0
        lax.fori_loop(0, B // 16, grp, 0, unroll=8)

    def pass1(idx_v, ob):
        def grp(g, _):
            vb = idx_v[pl.ds(g * 16, 16)] - _H0
            m = vb >= 0
            vals = plsc.load_gather(lane1, [jnp.maximum(vb, 0)])
            prev = ob[pl.ds(g * 16, 16)]
            ob[pl.ds(g * 16, 16)] = jnp.where(m, vals, prev)
            return 0
        lax.fori_loop(0, B // 16, grp, 0, unroll=8)

    fire_idx(0, idxA, siA).start()
    fire_idx(1, idxB, siB).start()
    fire_l0(0, sl0).start()
    fire_l1(0, sl1).start()
    fire_l1t(0, sl1).start()

    def body(k, _):
        f0 = 2 * k
        f1 = f0 + 1
        fire_idx(f0, idxA, siA).wait()

        @pl.when(k > 0)
        def _():
            fire_out(f0 - 2, obA, soA).wait()
        fire_l0(f0, sl0).wait()
        pass0(idxA, obA)
        fire_l0(f1, sl0).start()
        fire_l1(f0, sl1).wait()
        fire_l1t(f0, sl1).wait()
        pass1(idxA, obA)
        fire_l1(f1, sl1).start()
        fire_l1t(f1, sl1).start()
        fire_out(f0, obA, soA).start()

        @pl.when(k < _NPAIR - 1)
        def _():
            fire_idx(f0 + 2, idxA, siA).start()
        fire_idx(f1, idxB, siB).wait()

        @pl.when(k > 0)
        def _():
            fire_out(f1 - 2, obB, soB).wait()
        fire_l0(f1, sl0).wait()
        pass0(idxB, obB)

        @pl.when(k < _NPAIR - 1)
        def _():
            fire_l0(f0 + 2, sl0).start()
        fire_l1(f1, sl1).wait()
        fire_l1t(f1, sl1).wait()
        pass1(idxB, obB)

        @pl.when(k < _NPAIR - 1)
        def _():
            fire_l1(f0 + 2, sl1).start()
            fire_l1t(f0 + 2, sl1).start()
        fire_out(f1, obB, soB).start()

        @pl.when(k < _NPAIR - 1)
        def _():
            fire_idx(f1 + 2, idxB, siB).start()
        return 0

    lax.fori_loop(0, _NPAIR, body, 0, unroll=False)
    fire_out(F - 2, obA, soA).wait()
    fire_out(F - 1, obB, soB).wait()


_gather_call = functools.partial(
    pl.kernel,
    mesh=plsc.VectorSubcoreMesh(core_axis_name="c", subcore_axis_name="s"),
    out_type=jax.ShapeDtypeStruct((F * D, B), jnp.float32),
    compiler_params=pltpu.CompilerParams(needs_layout_passes=False),
    scratch_types=[
        pltpu.VMEM((_H0,), jnp.float32),
        pltpu.VMEM((_H1A + 128,), jnp.float32),
        pltpu.VMEM((B,), jnp.int32),
        pltpu.VMEM((B,), jnp.int32),
        pltpu.VMEM((B,), jnp.float32),
        pltpu.VMEM((B,), jnp.float32),
        pltpu.SemaphoreType.DMA,
        pltpu.SemaphoreType.DMA,
        pltpu.SemaphoreType.DMA,
        pltpu.SemaphoreType.DMA,
        pltpu.SemaphoreType.DMA,
        pltpu.SemaphoreType.DMA,
    ],
)(_gather_body)


def _mlp_body(embT_ref, numT_ref, w0eT_ref, w0nT_ref, b0_ref, w1T_ref, b1_ref,
              w2T_ref, b2_ref, w3T_ref, b3_ref, out_ref):
    h = jnp.dot(w0eT_ref[...], embT_ref[...], preferred_element_type=jnp.float32)
    h = h + jnp.dot(w0nT_ref[...], numT_ref[...], preferred_element_type=jnp.float32)
    h = jnp.maximum(h + b0_ref[...], 0.0)
    h = jnp.maximum(jnp.dot(w1T_ref[...], h, preferred_element_type=jnp.float32) + b1_ref[...], 0.0)
    h = jnp.maximum(jnp.dot(w2T_ref[...], h, preferred_element_type=jnp.float32) + b2_ref[...], 0.0)
    out_ref[...] = jnp.dot(w3T_ref[...], h, preferred_element_type=jnp.float32) + b3_ref[...]


_BB = 512


def _mlp_call(embT, numT, w0eT, w0nT, b0, w1T, b1, w2T, b2, w3T, b3):
    full = lambda shape: pl.BlockSpec(shape, lambda i: (0, 0))
    return pl.pallas_call(
        _mlp_body,
        grid=(B // _BB,),
        in_specs=[
            pl.BlockSpec((F * D, _BB), lambda i: (0, i)),
            pl.BlockSpec((NUM, _BB), lambda i: (0, i)),
            full(w0eT.shape), full(w0nT.shape), full(b0.shape),
            full(w1T.shape), full(b1.shape),
            full(w2T.shape), full(b2.shape),
            full(w3T.shape), full(b3.shape),
        ],
        out_specs=pl.BlockSpec((1, _BB), lambda i: (0, i)),
        out_shape=jax.ShapeDtypeStruct((1, B), jnp.float32),
    )(embT, numT, w0eT, w0nT, b0, w1T, b1, w2T, b2, w3T, b3)


@jax.jit
def kernel(numerical_features, cat_features, tables, W0, b0, W1, b1, W2, b2, W3, b3):
    tt = tables.transpose(0, 2, 1)
    tail = jnp.pad(tt[:, :, _H0 + _H1A:], ((0, 0), (0, 0), (0, 128 - _H1T)))
    cidx = cat_features.T
    embT = _gather_call(tt, tail, cidx)
    outT = _mlp_call(
        embT, numerical_features.T,
        W0[:F * D].T, W0[F * D:].T, b0.reshape(-1, 1),
        W1.T, b1.reshape(-1, 1), W2.T, b2.reshape(-1, 1), W3.T, b3.reshape(-1, 1),
    )
    return outT.reshape(B, 1)


# speedup vs baseline: 5.7721x; 1.0035x over previous
import functools

import jax
import jax.numpy as jnp
from jax import lax
from jax.experimental import pallas as pl
from jax.experimental.pallas import tpu as pltpu
from jax.experimental.pallas import tpu_sc as plsc

B = 4096
F = 26
V = 100000
D = 32
NUM = 13

_info = plsc.get_sparse_core_info()
_NC, _NS = _info.num_cores, _info.num_subcores
_NW = _NC * _NS


_H0 = 50048
_H1 = V - _H0
_H1A = 49920
_H1T = _H1 - _H1A
_NPAIR = F // 2


def _gather_body(tt_hbm, tail_hbm, cidx_hbm, out_hbm,
                 lane0, lane1, idxA, idxB, obA, obB,
                 sl0, sl1, siA, siB, soA, soB):
    w = lax.axis_index("s") * _NC + lax.axis_index("c")

    def fire_idx(f, idx_v, sem):
        return pltpu.make_async_copy(cidx_hbm.at[f], idx_v, sem)

    def fire_l0(f, sem):
        return pltpu.make_async_copy(
            tt_hbm.at[f].at[w].at[pl.ds(0, _H0)], lane0, sem)

    def fire_l1(f, sem):
        return pltpu.make_async_copy(
            tt_hbm.at[f].at[w].at[pl.ds(_H0, _H1A)],
            lane1.at[pl.ds(0, _H1A)], sem)

    def fire_l1t(f, sem):
        return pltpu.make_async_copy(
            tail_hbm.at[f].at[w], lane1.at[pl.ds(_H1A, 128)], sem)

    def fire_out(f, ob, sem):
        return pltpu.make_async_copy(ob, out_hbm.at[f * D + w], sem)

    def pass0(idx_v, ob):
        def grp(g, _):
            vb = idx_v[pl.ds(g * 16, 16)]
            m = vb < _H0
            vals = plsc.load_gather(lane0, [jnp.minimum(vb, _H0 - 1)])
            ob[pl.ds(g * 16, 16)] = jnp.where(m, vals, 0.0)
            return 0
        lax.fori_loop(0, B // 16, grp, 0, unroll=8)

    def pass1(idx_v, ob):
        def grp(g, _):
            vb = idx_v[pl.ds(g * 16, 16)] - _H0
            m = vb >= 0
            vals = plsc.load_gather(lane1, [jnp.maximum(vb, 0)])
            prev = ob[pl.ds(g * 16, 16)]
            ob[pl.ds(g * 16, 16)] = jnp.where(m, vals, prev)
            return 0
        lax.fori_loop(0, B // 16, grp, 0, unroll=8)

    fire_idx(0, idxA, siA).start()
    fire_idx(1, idxB, siB).start()
    fire_l0(0, sl0).start()
    fire_l1(0, sl1).start()
    fire_l1t(0, sl1).start()

    def body(k, _):
        f0 = 2 * k
        f1 = f0 + 1
        fire_idx(f0, idxA, siA).wait()

        @pl.when(k > 0)
        def _():
            fire_out(f0 - 2, obA, soA).wait()
        fire_l0(f0, sl0).wait()
        pass0(idxA, obA)
        fire_l0(f1, sl0).start()
        fire_l1(f0, sl1).wait()
        fire_l1t(f0, sl1).wait()
        pass1(idxA, obA)
        fire_l1(f1, sl1).start()
        fire_l1t(f1, sl1).start()
        fire_out(f0, obA, soA).start()

        @pl.when(k < _NPAIR - 1)
        def _():
            fire_idx(f0 + 2, idxA, siA).start()
        fire_idx(f1, idxB, siB).wait()

        @pl.when(k > 0)
        def _():
            fire_out(f1 - 2, obB, soB).wait()
        fire_l0(f1, sl0).wait()
        pass0(idxB, obB)

        @pl.when(k < _NPAIR - 1)
        def _():
            fire_l0(f0 + 2, sl0).start()
        fire_l1(f1, sl1).wait()
        fire_l1t(f1, sl1).wait()
        pass1(idxB, obB)

        @pl.when(k < _NPAIR - 1)
        def _():
            fire_l1(f0 + 2, sl1).start()
            fire_l1t(f0 + 2, sl1).start()
        fire_out(f1, obB, soB).start()

        @pl.when(k < _NPAIR - 1)
        def _():
            fire_idx(f1 + 2, idxB, siB).start()
        return 0

    lax.fori_loop(0, _NPAIR, body, 0, unroll=False)
    fire_out(F - 2, obA, soA).wait()
    fire_out(F - 1, obB, soB).wait()


_gather_call = functools.partial(
    pl.kernel,
    mesh=plsc.VectorSubcoreMesh(core_axis_name="c", subcore_axis_name="s"),
    out_type=jax.ShapeDtypeStruct((F * D, B), jnp.float32),
    compiler_params=pltpu.CompilerParams(needs_layout_passes=False),
    scratch_types=[
        pltpu.VMEM((_H0,), jnp.float32),
        pltpu.VMEM((_H1A + 128,), jnp.float32),
        pltpu.VMEM((B,), jnp.int32),
        pltpu.VMEM((B,), jnp.int32),
        pltpu.VMEM((B,), jnp.float32),
        pltpu.VMEM((B,), jnp.float32),
        pltpu.SemaphoreType.DMA,
        pltpu.SemaphoreType.DMA,
        pltpu.SemaphoreType.DMA,
        pltpu.SemaphoreType.DMA,
        pltpu.SemaphoreType.DMA,
        pltpu.SemaphoreType.DMA,
    ],
)(_gather_body)


def _mlp_body(embT_ref, numT_ref, w0eT_ref, w0nT_ref, b0_ref, w1T_ref, b1_ref,
              w2T_ref, b2_ref, w3T_ref, b3_ref, out_ref):
    bf = jnp.bfloat16
    h = jnp.dot(w0eT_ref[...], embT_ref[...].astype(bf),
                preferred_element_type=jnp.float32)
    h = h + jnp.dot(w0nT_ref[...], numT_ref[...], preferred_element_type=jnp.float32)
    h = jnp.maximum(h + b0_ref[...], 0.0)
    h = jnp.maximum(jnp.dot(w1T_ref[...], h.astype(bf), preferred_element_type=jnp.float32) + b1_ref[...], 0.0)
    h = jnp.maximum(jnp.dot(w2T_ref[...], h.astype(bf), preferred_element_type=jnp.float32) + b2_ref[...], 0.0)
    out_ref[...] = jnp.dot(w3T_ref[...], h.astype(bf), preferred_element_type=jnp.float32) + b3_ref[...]


_BB = 512


def _mlp_call(embT, numT, w0eT, w0nT, b0, w1T, b1, w2T, b2, w3T, b3):
    full = lambda shape: pl.BlockSpec(shape, lambda i: (0, 0))
    return pl.pallas_call(
        _mlp_body,
        grid=(B // _BB,),
        in_specs=[
            pl.BlockSpec((F * D, _BB), lambda i: (0, i)),
            pl.BlockSpec((NUM, _BB), lambda i: (0, i)),
            full(w0eT.shape), full(w0nT.shape), full(b0.shape),
            full(w1T.shape), full(b1.shape),
            full(w2T.shape), full(b2.shape),
            full(w3T.shape), full(b3.shape),
        ],
        out_specs=pl.BlockSpec((1, _BB), lambda i: (0, i)),
        out_shape=jax.ShapeDtypeStruct((1, B), jnp.float32),
    )(embT, numT, w0eT, w0nT, b0, w1T, b1, w2T, b2, w3T, b3)


@jax.jit
def kernel(numerical_features, cat_features, tables, W0, b0, W1, b1, W2, b2, W3, b3):
    tt = tables.transpose(0, 2, 1)
    tail = jnp.pad(tt[:, :, _H0 + _H1A:], ((0, 0), (0, 0), (0, 128 - _H1T)))
    cidx = cat_features.T
    embT = _gather_call(tt, tail, cidx)
    bf = jnp.bfloat16
    outT = _mlp_call(
        embT, numerical_features.T.astype(bf),
        W0[:F * D].T.astype(bf), W0[F * D:].T.astype(bf), b0.reshape(-1, 1),
        W1.T.astype(bf), b1.reshape(-1, 1), W2.T.astype(bf), b2.reshape(-1, 1),
        W3.T.astype(bf), b3.reshape(-1, 1),
    )
    return outT.reshape(B, 1)


# speedup vs baseline: 5.8027x; 1.0053x over previous
import functools

import jax
import jax.numpy as jnp
from jax import lax
from jax.experimental import pallas as pl
from jax.experimental.pallas import tpu as pltpu
from jax.experimental.pallas import tpu_sc as plsc

B = 4096
F = 26
V = 100000
D = 32
NUM = 13

_info = plsc.get_sparse_core_info()
_NC, _NS = _info.num_cores, _info.num_subcores
_NW = _NC * _NS


_H0 = 50048
_H1 = V - _H0
_H1A = 49920
_H1T = _H1 - _H1A
_NPAIR = F // 2


def _gather_body(tt_hbm, tail_hbm, cidx_hbm, out_hbm,
                 lane0, lane1, idxA, idxB, obA, obB,
                 sl0, sl1, siA, siB, soA, soB):
    w = lax.axis_index("c") * _NS + lax.axis_index("s")

    def fire_idx(f, idx_v, sem):
        return pltpu.make_async_copy(cidx_hbm.at[f], idx_v, sem)

    def fire_l0(f, sem):
        return pltpu.make_async_copy(
            tt_hbm.at[f].at[w].at[pl.ds(0, _H0)], lane0, sem)

    def fire_l1(f, sem):
        return pltpu.make_async_copy(
            tt_hbm.at[f].at[w].at[pl.ds(_H0, _H1A)],
            lane1.at[pl.ds(0, _H1A)], sem)

    def fire_l1t(f, sem):
        return pltpu.make_async_copy(
            tail_hbm.at[f].at[w], lane1.at[pl.ds(_H1A, 128)], sem)

    def fire_out(f, ob, sem):
        return pltpu.make_async_copy(ob, out_hbm.at[f * D + w], sem)

    def pass0(idx_v, ob):
        def grp(g, _):
            vb = idx_v[pl.ds(g * 16, 16)]
            m = vb < _H0
            vals = plsc.load_gather(lane0, [jnp.minimum(vb, _H0 - 1)])
            ob[pl.ds(g * 16, 16)] = jnp.where(m, vals, 0.0)
            return 0
        lax.fori_loop(0, B // 16, grp, 0, unroll=8)

    def pass1(idx_v, ob):
        def grp(g, _):
            vb = idx_v[pl.ds(g * 16, 16)] - _H0
            m = vb >= 0
            vals = plsc.load_gather(lane1, [jnp.maximum(vb, 0)])
            prev = ob[pl.ds(g * 16, 16)]
            ob[pl.ds(g * 16, 16)] = jnp.where(m, vals, prev)
            return 0
        lax.fori_loop(0, B // 16, grp, 0, unroll=8)

    fire_idx(0, idxA, siA).start()
    fire_idx(1, idxB, siB).start()
    fire_l0(0, sl0).start()
    fire_l1(0, sl1).start()
    fire_l1t(0, sl1).start()

    def body(k, _):
        f0 = 2 * k
        f1 = f0 + 1
        fire_idx(f0, idxA, siA).wait()

        @pl.when(k > 0)
        def _():
            fire_out(f0 - 2, obA, soA).wait()
        fire_l0(f0, sl0).wait()
        pass0(idxA, obA)
        fire_l0(f1, sl0).start()
        fire_l1(f0, sl1).wait()
        fire_l1t(f0, sl1).wait()
        pass1(idxA, obA)
        fire_l1(f1, sl1).start()
        fire_l1t(f1, sl1).start()
        fire_out(f0, obA, soA).start()

        @pl.when(k < _NPAIR - 1)
        def _():
            fire_idx(f0 + 2, idxA, siA).start()
        fire_idx(f1, idxB, siB).wait()

        @pl.when(k > 0)
        def _():
            fire_out(f1 - 2, obB, soB).wait()
        fire_l0(f1, sl0).wait()
        pass0(idxB, obB)

        @pl.when(k < _NPAIR - 1)
        def _():
            fire_l0(f0 + 2, sl0).start()
        fire_l1(f1, sl1).wait()
        fire_l1t(f1, sl1).wait()
        pass1(idxB, obB)

        @pl.when(k < _NPAIR - 1)
        def _():
            fire_l1(f0 + 2, sl1).start()
            fire_l1t(f0 + 2, sl1).start()
        fire_out(f1, obB, soB).start()

        @pl.when(k < _NPAIR - 1)
        def _():
            fire_idx(f1 + 2, idxB, siB).start()
        return 0

    lax.fori_loop(0, _NPAIR, body, 0, unroll=False)
    fire_out(F - 2, obA, soA).wait()
    fire_out(F - 1, obB, soB).wait()


_gather_call = functools.partial(
    pl.kernel,
    mesh=plsc.VectorSubcoreMesh(core_axis_name="c", subcore_axis_name="s"),
    out_type=jax.ShapeDtypeStruct((F * D, B), jnp.float32),
    compiler_params=pltpu.CompilerParams(needs_layout_passes=False),
    scratch_types=[
        pltpu.VMEM((_H0,), jnp.float32),
        pltpu.VMEM((_H1A + 128,), jnp.float32),
        pltpu.VMEM((B,), jnp.int32),
        pltpu.VMEM((B,), jnp.int32),
        pltpu.VMEM((B,), jnp.float32),
        pltpu.VMEM((B,), jnp.float32),
        pltpu.SemaphoreType.DMA,
        pltpu.SemaphoreType.DMA,
        pltpu.SemaphoreType.DMA,
        pltpu.SemaphoreType.DMA,
        pltpu.SemaphoreType.DMA,
        pltpu.SemaphoreType.DMA,
    ],
)(_gather_body)


def _mlp_body(embT_ref, numT_ref, w0eT_ref, w0nT_ref, b0_ref, w1T_ref, b1_ref,
              w2T_ref, b2_ref, w3T_ref, b3_ref, out_ref):
    bf = jnp.bfloat16
    h = jnp.dot(w0eT_ref[...], embT_ref[...].astype(bf),
                preferred_element_type=jnp.float32)
    h = h + jnp.dot(w0nT_ref[...], numT_ref[...], preferred_element_type=jnp.float32)
    h = jnp.maximum(h + b0_ref[...], 0.0)
    h = jnp.maximum(jnp.dot(w1T_ref[...], h.astype(bf), preferred_element_type=jnp.float32) + b1_ref[...], 0.0)
    h = jnp.maximum(jnp.dot(w2T_ref[...], h.astype(bf), preferred_element_type=jnp.float32) + b2_ref[...], 0.0)
    out_ref[...] = jnp.dot(w3T_ref[...], h.astype(bf), preferred_element_type=jnp.float32) + b3_ref[...]


_BB = 512


def _mlp_call(embT, numT, w0eT, w0nT, b0, w1T, b1, w2T, b2, w3T, b3):
    full = lambda shape: pl.BlockSpec(shape, lambda i: (0, 0))
    return pl.pallas_call(
        _mlp_body,
        grid=(B // _BB,),
        in_specs=[
            pl.BlockSpec((F * D, _BB), lambda i: (0, i)),
            pl.BlockSpec((NUM, _BB), lambda i: (0, i)),
            full(w0eT.shape), full(w0nT.shape), full(b0.shape),
            full(w1T.shape), full(b1.shape),
            full(w2T.shape), full(b2.shape),
            full(w3T.shape), full(b3.shape),
        ],
        out_specs=pl.BlockSpec((1, _BB), lambda i: (0, i)),
        out_shape=jax.ShapeDtypeStruct((1, B), jnp.float32),
    )(embT, numT, w0eT, w0nT, b0, w1T, b1, w2T, b2, w3T, b3)


@jax.jit
def kernel(numerical_features, cat_features, tables, W0, b0, W1, b1, W2, b2, W3, b3):
    tt = tables.transpose(0, 2, 1)
    tail = jnp.pad(tt[:, :, _H0 + _H1A:], ((0, 0), (0, 0), (0, 128 - _H1T)))
    cidx = cat_features.T
    embT = _gather_call(tt, tail, cidx)
    bf = jnp.bfloat16
    outT = _mlp_call(
        embT, numerical_features.T.astype(bf),
        W0[:F * D].T.astype(bf), W0[F * D:].T.astype(bf), b0.reshape(-1, 1),
        W1.T.astype(bf), b1.reshape(-1, 1), W2.T.astype(bf), b2.reshape(-1, 1),
        W3.T.astype(bf), b3.reshape(-1, 1),
    )
    return outT.reshape(B, 1)
